# Initial kernel scaffold; baseline (speedup 1.0000x reference)
import jax
import jax.numpy as jnp
from jax.experimental import pallas as pl


def kernel(input, point_index, sample_points, sample_inverse_sigmas):
    raise NotImplementedError("write your pallas kernel here")


# speedup vs baseline: 1.6844x; 1.6844x over previous
import functools

import jax
import jax.numpy as jnp
from jax import lax
from jax.experimental import pallas as pl
from jax.experimental.pallas import tpu as pltpu
from jax.experimental.pallas import tpu_sc as plsc

SEG = 100000
Q = 16
NC, NS = 2, 16
NW = NC * NS
C = 1024
CHUNKS = 98
PPW = C * CHUNKS
NPAD = NW * PPW
SLAB = 6256
ROWS_LAST = SEG - (NS - 1) * SLAB


def _sc_body(x_hbm, idx_hbm, prm_hbm, out_hbm, xb, ib, tb, pb, acc, sem):
    c = lax.axis_index("c")
    s = lax.axis_index("s")
    wid = s * NC + c

    pltpu.sync_copy(prm_hbm, pb)
    p0 = pb[0]
    p1 = pb[1]
    s0 = pb[2]
    s1 = pb[3]

    zeros = jnp.zeros((Q,), jnp.float32)

    @plsc.parallel_loop(0, C)
    def _(i):
        tb[i] = zeros

    slab = pl.multiple_of(s * SLAB, 16)

    def zero_rows(nrows):
        for z in range(nrows // C):
            pltpu.sync_copy(tb, acc.at[pl.ds(slab + z * C, C)])
        rem = nrows % C
        if rem:
            pltpu.sync_copy(tb.at[pl.ds(0, rem)],
                            acc.at[pl.ds(slab + (nrows // C) * C, rem)])

    @pl.when(s < NS - 1)
    def _():
        zero_rows(SLAB)

    @pl.when(s == NS - 1)
    def _():
        zero_rows(ROWS_LAST)

    plsc.subcore_barrier()

    def chunk_body(j, _):
        base = wid * PPW + j * C
        pltpu.sync_copy(x_hbm.at[pl.ds(pl.multiple_of(2 * base, 256), 2 * C)],
                        xb)
        pltpu.sync_copy(
            idx_hbm.at[pl.ds(pl.multiple_of(base // 128, 8), C // 128)], ib)

        @plsc.parallel_loop(0, C // 8, unroll=2)
        def _(k):
            xv = xb[pl.ds(16 * k, 16)]
            for u in range(8):
                x0 = xv[2 * u]
                x1 = xv[2 * u + 1]
                z0 = (x0 - p0) * s0
                z1 = (x1 - p1) * s1
                tb[8 * k + u] = jnp.exp(-(z0 * z0 + z1 * z1))

        copies = [
            pltpu.async_copy(tb.at[pl.ds(jj * 128, 128)], acc.at[ib.at[jj]],
                             sem, add=True)
            for jj in range(C // 128)
        ]
        for cp in copies:
            cp.wait()
        return None

    lax.fori_loop(0, CHUNKS, chunk_body, None)

    plsc.subcore_barrier()

    @pl.when(s < NS - 1)
    def _():
        pltpu.sync_copy(acc.at[pl.ds(slab, SLAB)],
                        out_hbm.at[c, pl.ds(slab, SLAB)])

    @pl.when(s == NS - 1)
    def _():
        pltpu.sync_copy(acc.at[pl.ds(slab, ROWS_LAST)],
                        out_hbm.at[c, pl.ds(slab, ROWS_LAST)])


_sc_kernel = functools.partial(
    pl.kernel,
    out_type=jax.ShapeDtypeStruct((NC, SEG, Q), jnp.float32),
    mesh=plsc.VectorSubcoreMesh(core_axis_name="c", subcore_axis_name="s"),
    scratch_types=[
        pltpu.VMEM((2 * C,), jnp.float32),
        pltpu.VMEM((C // 128, 128), jnp.int32),
        pltpu.VMEM((C, Q), jnp.float32),
        pltpu.VMEM((4, Q), jnp.float32),
        pltpu.VMEM_SHARED((SEG, Q), jnp.float32),
        pltpu.SemaphoreType.DMA,
    ],
    compiler_params=pltpu.CompilerParams(use_tc_tiling_on_sc=False),
)(_sc_body)


def _combine_body(a_ref, o_ref):
    o_ref[...] = a_ref[0] + a_ref[1]


_BLK = 2000


def _combine(partials):
    return pl.pallas_call(
        _combine_body,
        grid=(SEG // _BLK,),
        in_specs=[pl.BlockSpec((NC, _BLK, Q), lambda i: (0, i, 0))],
        out_specs=pl.BlockSpec((_BLK, Q), lambda i: (i, 0)),
        out_shape=jax.ShapeDtypeStruct((SEG, Q), jnp.float32),
    )(partials)


def kernel(input, point_index, sample_points, sample_inverse_sigmas):
    pad = NPAD - input.shape[0]
    xp = jnp.concatenate(
        [input, jnp.full((pad, 2), 1e4, jnp.float32)], axis=0).reshape(-1)
    ip = jnp.concatenate(
        [point_index.astype(jnp.int32), jnp.zeros((pad,), jnp.int32)])
    ip = ip.reshape(NPAD // 128, 128)
    prm = jnp.concatenate(
        [sample_points.astype(jnp.float32),
         sample_inverse_sigmas.astype(jnp.float32)], axis=0)
    partials = _sc_kernel(xp, ip, prm)
    return _combine(partials)


# speedup vs baseline: 2.0376x; 1.2097x over previous
import functools

import jax
import jax.numpy as jnp
from jax import lax
from jax.experimental import pallas as pl
from jax.experimental.pallas import tpu as pltpu
from jax.experimental.pallas import tpu_sc as plsc

N = 3200000
SEG = 100000
Q = 16
NC, NS = 2, 16
NW = NC * NS
PPW = N // NW
C = 1024
CHUNKS = PPW // C
TAIL = PPW - CHUNKS * C
SLAB = 6256
ROWS_LAST = SEG - (NS - 1) * SLAB


def _sc_body(x_hbm, idx_hbm, prm_hbm, out_hbm, xb, ib, tb, pb, acc, sem):
    c = lax.axis_index("c")
    s = lax.axis_index("s")
    wid = s * NC + c

    pltpu.sync_copy(prm_hbm, pb)
    p0 = pb[0]
    p1 = pb[1]
    s0 = pb[2]
    s1 = pb[3]

    zeros = jnp.zeros((Q,), jnp.float32)

    @plsc.parallel_loop(0, C)
    def _(i):
        tb[i] = zeros

    slab = pl.multiple_of(s * SLAB, 16)

    def zero_rows(nrows):
        for z in range(nrows // C):
            pltpu.sync_copy(tb, acc.at[pl.ds(slab + z * C, C)])
        rem = nrows % C
        if rem:
            pltpu.sync_copy(tb.at[pl.ds(0, rem)],
                            acc.at[pl.ds(slab + (nrows // C) * C, rem)])

    @pl.when(s < NS - 1)
    def _():
        zero_rows(SLAB)

    @pl.when(s == NS - 1)
    def _():
        zero_rows(ROWS_LAST)

    plsc.subcore_barrier()

    def do_chunk(base, n):
        pltpu.sync_copy(x_hbm.at[pl.ds(pl.multiple_of(2 * base, 64), 2 * n)],
                        xb.at[pl.ds(0, 2 * n)])
        pltpu.sync_copy(idx_hbm.at[pl.ds(pl.multiple_of(base, 32), n)],
                        ib.at[pl.ds(0, n)])

        @plsc.parallel_loop(0, n // 8, unroll=2)
        def _(k):
            xv = xb[pl.ds(16 * k, 16)]
            for u in range(8):
                x0 = xv[2 * u]
                x1 = xv[2 * u + 1]
                z0 = (x0 - p0) * s0
                z1 = (x1 - p1) * s1
                tb[8 * k + u] = jnp.exp(-(z0 * z0 + z1 * z1))

        copies = []
        off = 0
        while off < n:
            b = min(128, n - off)
            copies.append(
                pltpu.async_copy(tb.at[pl.ds(off, b)],
                                 acc.at[ib.at[pl.ds(off, b)]], sem, add=True))
            off += b
        for cp in copies:
            cp.wait()

    base0 = wid * PPW

    def chunk_body(j, _):
        do_chunk(base0 + j * C, C)
        return None

    lax.fori_loop(0, CHUNKS, chunk_body, None)
    if TAIL:
        do_chunk(base0 + CHUNKS * C, TAIL)

    plsc.subcore_barrier()

    @pl.when(s < NS - 1)
    def _():
        pltpu.sync_copy(acc.at[pl.ds(slab, SLAB)],
                        out_hbm.at[c, pl.ds(slab, SLAB)])

    @pl.when(s == NS - 1)
    def _():
        pltpu.sync_copy(acc.at[pl.ds(slab, ROWS_LAST)],
                        out_hbm.at[c, pl.ds(slab, ROWS_LAST)])


_sc_kernel = functools.partial(
    pl.kernel,
    out_type=jax.ShapeDtypeStruct((NC, SEG, Q), jnp.float32),
    mesh=plsc.VectorSubcoreMesh(core_axis_name="c", subcore_axis_name="s"),
    scratch_types=[
        pltpu.VMEM((2 * C,), jnp.float32),
        pltpu.VMEM((C,), jnp.int32),
        pltpu.VMEM((C, Q), jnp.float32),
        pltpu.VMEM((4, Q), jnp.float32),
        pltpu.VMEM_SHARED((SEG, Q), jnp.float32),
        pltpu.SemaphoreType.DMA,
    ],
    compiler_params=pltpu.CompilerParams(use_tc_tiling_on_sc=False),
)(_sc_body)


def _combine_body(a_ref, o_ref):
    o_ref[...] = a_ref[0] + a_ref[1]


_BLK = 2000


def _combine(partials):
    return pl.pallas_call(
        _combine_body,
        grid=(SEG // _BLK,),
        in_specs=[pl.BlockSpec((NC, _BLK, Q), lambda i: (0, i, 0))],
        out_specs=pl.BlockSpec((_BLK, Q), lambda i: (i, 0)),
        out_shape=jax.ShapeDtypeStruct((SEG, Q), jnp.float32),
    )(partials)


def kernel(input, point_index, sample_points, sample_inverse_sigmas):
    xflat = input.reshape(-1)
    idx = point_index.astype(jnp.int32)
    prm = jnp.concatenate(
        [sample_points.astype(jnp.float32),
         sample_inverse_sigmas.astype(jnp.float32)], axis=0)
    partials = _sc_kernel(xflat, idx, prm)
    return _combine(partials)


# speedup vs baseline: 14.8445x; 7.2854x over previous
import functools

import jax
import jax.numpy as jnp
from jax import lax
from jax.experimental import pallas as pl
from jax.experimental.pallas import tpu as pltpu
from jax.experimental.pallas import tpu_sc as plsc

N = 3200000
SEG = 100000
Q = 16
NC, NS = 2, 16
NW = NC * NS
B = 128
NBLK = N // B
BLK_LO = NBLK // NW
EXTRA = NBLK - BLK_LO * NW
CB = 8
CHUNKS = BLK_LO // CB
TAIL_LO = BLK_LO - CHUNKS * CB
TAIL_HI = TAIL_LO + 1
C = CB * B
SLAB = 6256
ROWS_LAST = SEG - (NS - 1) * SLAB


def _sc_body(x_hbm, idx_hbm, prm_hbm, out_hbm, xb, ib, tb, pb, acc, sem):
    c = lax.axis_index("c")
    s = lax.axis_index("s")
    wid = s * NC + c

    pltpu.sync_copy(prm_hbm, pb)
    p0 = pb[0]
    p1 = pb[1]
    s0 = pb[2]
    s1 = pb[3]

    zeros = jnp.zeros((Q,), jnp.float32)

    @plsc.parallel_loop(0, C)
    def _(i):
        tb[i] = zeros

    slab = pl.multiple_of(s * SLAB, 16)

    def zero_rows(nrows):
        for z in range(nrows // C):
            pltpu.sync_copy(tb, acc.at[pl.ds(slab + z * C, C)])
        rem = nrows % C
        if rem:
            pltpu.sync_copy(tb.at[pl.ds(0, rem)],
                            acc.at[pl.ds(slab + (nrows // C) * C, rem)])

    @pl.when(s < NS - 1)
    def _():
        zero_rows(SLAB)

    @pl.when(s == NS - 1)
    def _():
        zero_rows(ROWS_LAST)

    plsc.subcore_barrier()

    def do_chunk(blk, nb):
        n = nb * B
        pltpu.sync_copy(x_hbm.at[pl.ds(blk, nb)], xb.at[pl.ds(0, nb)])
        pltpu.sync_copy(idx_hbm.at[pl.ds(pl.multiple_of(blk * B, B), n)],
                        ib.at[pl.ds(0, n)])

        @plsc.parallel_loop(0, nb * 8, unroll=2)
        def _(k):
            g = k // 8
            kk = k % 8
            xv0 = xb[g, 0, pl.ds(16 * kk, 16)]
            xv1 = xb[g, 1, pl.ds(16 * kk, 16)]
            for u in range(16):
                z0 = (xv0[u] - p0) * s0
                z1 = (xv1[u] - p1) * s1
                tb[128 * g + 16 * kk + u] = jnp.exp(-(z0 * z0 + z1 * z1))

        copies = [
            pltpu.async_copy(tb.at[pl.ds(jj * B, B)],
                             acc.at[ib.at[pl.ds(jj * B, B)]], sem, add=True)
            for jj in range(nb)
        ]
        for cp in copies:
            cp.wait()

    blk0 = BLK_LO * wid + jnp.minimum(wid, EXTRA)

    def chunk_body(j, _):
        do_chunk(blk0 + j * CB, CB)
        return None

    lax.fori_loop(0, CHUNKS, chunk_body, None)

    @pl.when(wid < EXTRA)
    def _():
        do_chunk(blk0 + CHUNKS * CB, TAIL_HI)

    @pl.when(wid >= EXTRA)
    def _():
        do_chunk(blk0 + CHUNKS * CB, TAIL_LO)

    plsc.subcore_barrier()

    @pl.when(s < NS - 1)
    def _():
        pltpu.sync_copy(acc.at[pl.ds(slab, SLAB)],
                        out_hbm.at[c, pl.ds(slab, SLAB)])

    @pl.when(s == NS - 1)
    def _():
        pltpu.sync_copy(acc.at[pl.ds(slab, ROWS_LAST)],
                        out_hbm.at[c, pl.ds(slab, ROWS_LAST)])


_sc_kernel = functools.partial(
    pl.kernel,
    out_type=jax.ShapeDtypeStruct((NC, SEG, Q), jnp.float32),
    mesh=plsc.VectorSubcoreMesh(core_axis_name="c", subcore_axis_name="s"),
    scratch_types=[
        pltpu.VMEM((CB, 2, B), jnp.float32),
        pltpu.VMEM((C,), jnp.int32),
        pltpu.VMEM((C, Q), jnp.float32),
        pltpu.VMEM((4, Q), jnp.float32),
        pltpu.VMEM_SHARED((SEG, Q), jnp.float32),
        pltpu.SemaphoreType.DMA,
    ],
    compiler_params=pltpu.CompilerParams(use_tc_tiling_on_sc=False),
)(_sc_body)


def _combine_body(a_ref, o_ref):
    o_ref[...] = a_ref[0] + a_ref[1]


_BLK = 2000


def _combine(partials):
    return pl.pallas_call(
        _combine_body,
        grid=(SEG // _BLK,),
        in_specs=[pl.BlockSpec((NC, _BLK, Q), lambda i: (0, i, 0))],
        out_specs=pl.BlockSpec((_BLK, Q), lambda i: (i, 0)),
        out_shape=jax.ShapeDtypeStruct((SEG, Q), jnp.float32),
    )(partials)


def kernel(input, point_index, sample_points, sample_inverse_sigmas):
    xview = input.reshape(NBLK, B, 2).transpose(0, 2, 1)
    idx = point_index.astype(jnp.int32)
    prm = jnp.concatenate(
        [sample_points.astype(jnp.float32),
         sample_inverse_sigmas.astype(jnp.float32)], axis=0)
    partials = _sc_kernel(xview, idx, prm)
    return _combine(partials)
